# baseline (speedup 1.0000x reference)
import jax
import jax.numpy as jnp
import numpy as np
from jax.experimental import pallas as pl
from jax.experimental.pallas import tpu as pltpu

B_SZ = 4096
D = 768
H = 4
OC1 = 192
BT = 64
HB = BT // 2

_EDGES = [(0, 1, 0), (1, 0, 0), (0, 2, 0), (2, 0, 0), (0, 3, 0), (3, 0, 0),
          (1, 2, 1), (2, 1, 1), (1, 3, 1), (3, 1, 1), (2, 3, 1), (3, 2, 1)]
_INC = {d: [(s, wk) for (s, dd, wk) in _EDGES if dd == d] for d in range(4)}


def _bdot(a, b):
    return jax.lax.dot_general(a, b, (((1,), (0,)), ((), ())),
                               preferred_element_type=jnp.float32)


def _split(a):
    hi = a.astype(jnp.bfloat16)
    lo = (a - hi.astype(jnp.float32)).astype(jnp.bfloat16)
    return hi, lo


def _dot3(a, b_hi, b_lo):
    a_hi, a_lo = _split(a)
    return (_bdot(a_hi, b_hi) + _bdot(a_lo, b_hi)) + _bdot(a_hi, b_lo)


def _softmax3(logits):
    amax = jnp.maximum(jnp.maximum(logits[0], logits[1]), logits[2])
    exps = [jnp.exp(l - amax) for l in logits]
    den = exps[0] + exps[1] + exps[2] + 1e-16
    return [ex / den for ex in exps]


class _P:
    pass


def _attn1(p, xs, xlr):
    xln = [xlr[n * HB:(n + 1) * HB, 0:D] + p.bl1 for n in range(4)]
    yrn = [[xlr[n * HB:(n + 1) * HB, D:2 * D] + p.br1 + p.ea1[wk]
            for wk in range(2)] for n in range(4)]
    x2n = []
    for dn in range(4):
        logits, srcs = [], []
        for (s, wk) in _INC[dn]:
            m = xln[s] + yrn[dn][wk]
            m = jnp.where(m > 0, m, 0.2 * m)
            logits.append(_dot3(m, p.a1h, p.a1l))
            srcs.append(s)
        alphas = _softmax3(logits)
        outd = None
        for al, s in zip(alphas, srcs):
            a_hi, a_lo = _split(al)
            af = _bdot(a_hi, p.b1m) + _bdot(a_lo, p.b1m)
            t = af * xln[s]
            outd = t if outd is None else outd + t
        r = xs[dn] + outd + p.bias1
        x2n.append(jnp.where(r > 0, r, jnp.exp(r) - 1.0))
    return x2n


def _attn2(p, x2n, xlr2):
    acc = [None] * 4
    for h in range(4):
        o = h * D
        xl2n = [xlr2[n * HB:(n + 1) * HB, o:o + D] + p.bl2[h] for n in range(4)]
        yr2n = [[xlr2[n * HB:(n + 1) * HB, 4 * D + o:4 * D + o + D]
                 + p.br2[h] + p.ea2[wk][h] for wk in range(2)]
                for n in range(4)]
        att2h = p.att2[h]
        for dn in range(4):
            logits, srcs = [], []
            for (s, wk) in _INC[dn]:
                m = xl2n[s] + yr2n[dn][wk]
                m = jnp.where(m > 0, m, 0.2 * m)
                logits.append(jnp.sum(m * att2h, axis=1, keepdims=True))
                srcs.append(s)
            alphas = _softmax3(logits)
            outd = None
            for al, s in zip(alphas, srcs):
                t = al * xl2n[s]
                outd = t if outd is None else outd + t
            acc[dn] = outd if acc[dn] is None else acc[dn] + outd
    return jnp.concatenate(
        [x2n[dn] + 0.25 * acc[dn] + p.bias2 for dn in range(4)], axis=0)


def _mlp(p, x3):
    hmid = _dot3(x3, p.w1h, p.w1l) + p.b1v
    mu = jnp.mean(hmid, axis=1, keepdims=True)
    var = jnp.mean((hmid - mu) ** 2, axis=1, keepdims=True)
    hmid = (hmid - mu) * jax.lax.rsqrt(var + 1e-5) * p.ln_g + p.ln_b
    hmid = jnp.maximum(hmid, 0.0)
    return _dot3(hmid, p.w2h, p.w2l) + p.b2v


def _body(q_ref, p_ref, e_ref, qv_ref, te_ref,
          wlr1h_ref, wlr1l_ref, a1h_ref, a1l_ref, b1m_ref,
          wlr2h_ref, wlr2l_ref, w1h_ref, w1l_ref, w2h_ref, w2l_ref,
          v768_ref, v3072_ref, out_ref):
    p = _P()
    p.bl1 = v768_ref[0:1, :]
    p.br1 = v768_ref[1:2, :]
    p.ea1 = [v768_ref[2:3, :], v768_ref[3:4, :]]
    p.bias1 = v768_ref[5:6, :]
    p.bias2 = v768_ref[6:7, :]
    p.b1v = v768_ref[7:8, :]
    p.ln_g = v768_ref[8:9, :]
    p.ln_b = v768_ref[9:10, :]
    p.b2v = v768_ref[10:11, :]
    p.bl2 = [v3072_ref[0:1, h * D:(h + 1) * D] for h in range(4)]
    p.br2 = [v3072_ref[1:2, h * D:(h + 1) * D] for h in range(4)]
    p.ea2 = [[v3072_ref[2 + wk:3 + wk, h * D:(h + 1) * D] for h in range(4)]
             for wk in range(2)]
    p.att2 = [v3072_ref[4:5, h * D:(h + 1) * D] for h in range(4)]
    p.a1h = a1h_ref[...]
    p.a1l = a1l_ref[...]
    p.b1m = b1m_ref[...]
    p.w1h = w1h_ref[...]
    p.w1l = w1l_ref[...]
    p.w2h = w2h_ref[...]
    p.w2l = w2l_ref[...]
    te = te_ref[...]

    def build(lo):
        xs = [q_ref[lo:lo + HB] + te[0:1],
              p_ref[lo:lo + HB] + te[1:2],
              e_ref[lo:lo + HB] + te[2:3],
              qv_ref[lo:lo + HB] + te[3:4]]
        return xs, jnp.concatenate(xs, axis=0)

    wlr1h, wlr1l = wlr1h_ref[...], wlr1l_ref[...]
    wlr2h, wlr2l = wlr2h_ref[...], wlr2l_ref[...]

    xsA, xA = build(0)
    xsB, xB = build(HB)
    xlrA = _dot3(xA, wlr1h, wlr1l)
    xlrB = _dot3(xB, wlr1h, wlr1l)
    x2nA = _attn1(p, xsA, xlrA)
    xlr2A = _dot3(jnp.concatenate(x2nA, axis=0), wlr2h, wlr2l)
    x2nB = _attn1(p, xsB, xlrB)
    xlr2B = _dot3(jnp.concatenate(x2nB, axis=0), wlr2h, wlr2l)
    x3A = _attn2(p, x2nA, xlr2A)
    outA = _mlp(p, x3A)
    x3B = _attn2(p, x2nB, xlr2B)
    outB = _mlp(p, x3B)
    for n in range(4):
        out_ref[n, 0:HB, :] = outA[n * HB:(n + 1) * HB]
        out_ref[n, HB:BT, :] = outB[n * HB:(n + 1) * HB]


def kernel(q, p, e, qv, params):
    L = params['layers']
    w = jax.nn.sigmoid(params['weak_weight'])
    f32 = jnp.float32

    we1 = L[0]['We'].reshape(-1)
    att1 = L[0]['att'].reshape(-1)
    we2 = L[1]['We'].reshape(-1)
    att2 = L[1]['att'].reshape(-1)

    zeros = jnp.zeros((D,), f32)
    v768 = jnp.stack([
        L[0]['bl'], L[0]['br'], we1, w * we1, zeros,
        L[0]['bias'], L[1]['bias'], params['b1'], params['ln_g'],
        params['ln_b'], params['b2'], zeros, zeros, zeros, zeros, zeros,
    ]).astype(f32)
    z2 = jnp.zeros((4 * D,), f32)
    v3072 = jnp.stack([
        L[1]['bl'], L[1]['br'], we2, w * we2, att2, z2, z2, z2,
    ]).astype(f32)

    head_of_lane = jnp.arange(D) // OC1
    a1 = jnp.where(head_of_lane[:, None] == jnp.arange(128)[None, :],
                   att1[:, None], 0.0).astype(f32)
    b1m = (jnp.arange(128)[:, None] == head_of_lane[None, :]).astype(
        jnp.bfloat16)

    wlr1 = jnp.concatenate([L[0]['Wl'], L[0]['Wr']], axis=1)
    wlr2 = jnp.concatenate([L[1]['Wl'], L[1]['Wr']], axis=1)
    wlr1h, wlr1l = _split(wlr1)
    wlr2h, wlr2l = _split(wlr2)
    a1h, a1l = _split(a1)
    w1h, w1l = _split(params['W1'])
    w2h, w2l = _split(params['W2'])

    cspec = lambda shape: pl.BlockSpec(shape, lambda i: (0,) * len(shape))
    out = pl.pallas_call(
        _body,
        grid=(B_SZ // BT,),
        in_specs=[
            pl.BlockSpec((BT, D), lambda i: (i, 0)),
            pl.BlockSpec((BT, D), lambda i: (i, 0)),
            pl.BlockSpec((BT, D), lambda i: (i, 0)),
            pl.BlockSpec((BT, D), lambda i: (i, 0)),
            cspec((4, D)),
            cspec((D, 2 * D)),
            cspec((D, 2 * D)),
            cspec((D, 128)),
            cspec((D, 128)),
            cspec((128, D)),
            cspec((D, 8 * D)),
            cspec((D, 8 * D)),
            cspec((D, D)),
            cspec((D, D)),
            cspec((D, D)),
            cspec((D, D)),
            cspec((16, D)),
            cspec((8, 4 * D)),
        ],
        out_specs=pl.BlockSpec((4, BT, D), lambda i: (0, i, 0)),
        out_shape=jax.ShapeDtypeStruct((4, B_SZ, D), f32),
    )(q, p, e, qv, params['type_embed'],
      wlr1h, wlr1l, a1h, a1l, b1m,
      wlr2h, wlr2l, w1h, w1l, w2h, w2l,
      v768, v3072)
    return out.transpose(1, 0, 2)


# speedup vs baseline: 1.1436x; 1.1436x over previous
import jax
import jax.numpy as jnp
import numpy as np
from jax.experimental import pallas as pl
from jax.experimental.pallas import tpu as pltpu

B_SZ = 4096
D = 768
H = 4
OC1 = 192
BT = 128
HB = BT // 2

_EDGES = [(0, 1, 0), (1, 0, 0), (0, 2, 0), (2, 0, 0), (0, 3, 0), (3, 0, 0),
          (1, 2, 1), (2, 1, 1), (1, 3, 1), (3, 1, 1), (2, 3, 1), (3, 2, 1)]
_INC = {d: [(s, wk) for (s, dd, wk) in _EDGES if dd == d] for d in range(4)}


def _bdot(a, b):
    return jax.lax.dot_general(a, b, (((1,), (0,)), ((), ())),
                               preferred_element_type=jnp.float32)


def _split(a):
    hi = a.astype(jnp.bfloat16)
    lo = (a - hi.astype(jnp.float32)).astype(jnp.bfloat16)
    return hi, lo


def _dot3(a, b_hi, b_lo):
    a_hi, a_lo = _split(a)
    return (_bdot(a_hi, b_hi) + _bdot(a_lo, b_hi)) + _bdot(a_hi, b_lo)


def _softmax3(logits):
    amax = jnp.maximum(jnp.maximum(logits[0], logits[1]), logits[2])
    exps = [jnp.exp(l - amax) for l in logits]
    den = exps[0] + exps[1] + exps[2] + 1e-16
    return [ex / den for ex in exps]


class _P:
    pass


def _attn1(p, xs, xlr):
    xln = [xlr[n * HB:(n + 1) * HB, 0:D] + p.bl1 for n in range(4)]
    yrn = [[xlr[n * HB:(n + 1) * HB, D:2 * D] + p.br1 + p.ea1[wk]
            for wk in range(2)] for n in range(4)]
    x2n = []
    for dn in range(4):
        logits, srcs = [], []
        for (s, wk) in _INC[dn]:
            m = xln[s] + yrn[dn][wk]
            m = jnp.where(m > 0, m, 0.2 * m)
            logits.append(_dot3(m, p.a1h, p.a1l))
            srcs.append(s)
        alphas = _softmax3(logits)
        outd = None
        for al, s in zip(alphas, srcs):
            a_hi, a_lo = _split(al)
            af = _bdot(a_hi, p.b1m) + _bdot(a_lo, p.b1m)
            t = af * xln[s]
            outd = t if outd is None else outd + t
        r = xs[dn] + outd + p.bias1
        x2n.append(jnp.where(r > 0, r, jnp.exp(r) - 1.0))
    return x2n


def _attn2(p, x2n, xlr2):
    acc = [None] * 4
    for h in range(4):
        o = h * D
        xl2n = [xlr2[n * HB:(n + 1) * HB, o:o + D] + p.bl2[h] for n in range(4)]
        yr2n = [[xlr2[n * HB:(n + 1) * HB, 4 * D + o:4 * D + o + D]
                 + p.br2[h] + p.ea2[wk][h] for wk in range(2)]
                for n in range(4)]
        att2h = p.att2[h]
        for dn in range(4):
            logits, srcs = [], []
            for (s, wk) in _INC[dn]:
                m = xl2n[s] + yr2n[dn][wk]
                m = jnp.where(m > 0, m, 0.2 * m)
                logits.append(jnp.sum(m * att2h, axis=1, keepdims=True))
                srcs.append(s)
            alphas = _softmax3(logits)
            outd = None
            for al, s in zip(alphas, srcs):
                t = al * xl2n[s]
                outd = t if outd is None else outd + t
            acc[dn] = outd if acc[dn] is None else acc[dn] + outd
    return jnp.concatenate(
        [x2n[dn] + 0.25 * acc[dn] + p.bias2 for dn in range(4)], axis=0)


def _mlp(p, x3):
    hmid = _dot3(x3, p.w1h, p.w1l) + p.b1v
    mu = jnp.mean(hmid, axis=1, keepdims=True)
    var = jnp.mean((hmid - mu) ** 2, axis=1, keepdims=True)
    hmid = (hmid - mu) * jax.lax.rsqrt(var + 1e-5) * p.ln_g + p.ln_b
    hmid = jnp.maximum(hmid, 0.0)
    return _dot3(hmid, p.w2h, p.w2l) + p.b2v


def _body(q_ref, p_ref, e_ref, qv_ref, te_ref,
          wlr1h_ref, wlr1l_ref, a1h_ref, a1l_ref, b1m_ref,
          wlr2h_ref, wlr2l_ref, w1h_ref, w1l_ref, w2h_ref, w2l_ref,
          v768_ref, v3072_ref, out_ref):
    p = _P()
    p.bl1 = v768_ref[0:1, :]
    p.br1 = v768_ref[1:2, :]
    p.ea1 = [v768_ref[2:3, :], v768_ref[3:4, :]]
    p.bias1 = v768_ref[5:6, :]
    p.bias2 = v768_ref[6:7, :]
    p.b1v = v768_ref[7:8, :]
    p.ln_g = v768_ref[8:9, :]
    p.ln_b = v768_ref[9:10, :]
    p.b2v = v768_ref[10:11, :]
    p.bl2 = [v3072_ref[0:1, h * D:(h + 1) * D] for h in range(4)]
    p.br2 = [v3072_ref[1:2, h * D:(h + 1) * D] for h in range(4)]
    p.ea2 = [[v3072_ref[2 + wk:3 + wk, h * D:(h + 1) * D] for h in range(4)]
             for wk in range(2)]
    p.att2 = [v3072_ref[4:5, h * D:(h + 1) * D] for h in range(4)]
    p.a1h = a1h_ref[...]
    p.a1l = a1l_ref[...]
    p.b1m = b1m_ref[...]
    p.w1h = w1h_ref[...]
    p.w1l = w1l_ref[...]
    p.w2h = w2h_ref[...]
    p.w2l = w2l_ref[...]
    te = te_ref[...]

    def build(lo):
        xs = [q_ref[lo:lo + HB] + te[0:1],
              p_ref[lo:lo + HB] + te[1:2],
              e_ref[lo:lo + HB] + te[2:3],
              qv_ref[lo:lo + HB] + te[3:4]]
        return xs, jnp.concatenate(xs, axis=0)

    wlr1h, wlr1l = wlr1h_ref[...], wlr1l_ref[...]
    wlr2h, wlr2l = wlr2h_ref[...], wlr2l_ref[...]

    xsA, xA = build(0)
    xsB, xB = build(HB)
    xlrA = _dot3(xA, wlr1h, wlr1l)
    xlrB = _dot3(xB, wlr1h, wlr1l)
    x2nA = _attn1(p, xsA, xlrA)
    xlr2A = _dot3(jnp.concatenate(x2nA, axis=0), wlr2h, wlr2l)
    x2nB = _attn1(p, xsB, xlrB)
    xlr2B = _dot3(jnp.concatenate(x2nB, axis=0), wlr2h, wlr2l)
    x3A = _attn2(p, x2nA, xlr2A)
    outA = _mlp(p, x3A)
    x3B = _attn2(p, x2nB, xlr2B)
    outB = _mlp(p, x3B)
    for n in range(4):
        out_ref[n, 0:HB, :] = outA[n * HB:(n + 1) * HB]
        out_ref[n, HB:BT, :] = outB[n * HB:(n + 1) * HB]


def kernel(q, p, e, qv, params):
    L = params['layers']
    w = jax.nn.sigmoid(params['weak_weight'])
    f32 = jnp.float32

    we1 = L[0]['We'].reshape(-1)
    att1 = L[0]['att'].reshape(-1)
    we2 = L[1]['We'].reshape(-1)
    att2 = L[1]['att'].reshape(-1)

    zeros = jnp.zeros((D,), f32)
    v768 = jnp.stack([
        L[0]['bl'], L[0]['br'], we1, w * we1, zeros,
        L[0]['bias'], L[1]['bias'], params['b1'], params['ln_g'],
        params['ln_b'], params['b2'], zeros, zeros, zeros, zeros, zeros,
    ]).astype(f32)
    z2 = jnp.zeros((4 * D,), f32)
    v3072 = jnp.stack([
        L[1]['bl'], L[1]['br'], we2, w * we2, att2, z2, z2, z2,
    ]).astype(f32)

    head_of_lane = jnp.arange(D) // OC1
    a1 = jnp.where(head_of_lane[:, None] == jnp.arange(128)[None, :],
                   att1[:, None], 0.0).astype(f32)
    b1m = (jnp.arange(128)[:, None] == head_of_lane[None, :]).astype(
        jnp.bfloat16)

    wlr1 = jnp.concatenate([L[0]['Wl'], L[0]['Wr']], axis=1)
    wlr2 = jnp.concatenate([L[1]['Wl'], L[1]['Wr']], axis=1)
    wlr1h, wlr1l = _split(wlr1)
    wlr2h, wlr2l = _split(wlr2)
    a1h, a1l = _split(a1)
    w1h, w1l = _split(params['W1'])
    w2h, w2l = _split(params['W2'])

    cspec = lambda shape: pl.BlockSpec(shape, lambda i: (0,) * len(shape))
    out = pl.pallas_call(
        _body,
        grid=(B_SZ // BT,),
        in_specs=[
            pl.BlockSpec((BT, D), lambda i: (i, 0)),
            pl.BlockSpec((BT, D), lambda i: (i, 0)),
            pl.BlockSpec((BT, D), lambda i: (i, 0)),
            pl.BlockSpec((BT, D), lambda i: (i, 0)),
            cspec((4, D)),
            cspec((D, 2 * D)),
            cspec((D, 2 * D)),
            cspec((D, 128)),
            cspec((D, 128)),
            cspec((128, D)),
            cspec((D, 8 * D)),
            cspec((D, 8 * D)),
            cspec((D, D)),
            cspec((D, D)),
            cspec((D, D)),
            cspec((D, D)),
            cspec((16, D)),
            cspec((8, 4 * D)),
        ],
        out_specs=pl.BlockSpec((4, BT, D), lambda i: (0, i, 0)),
        out_shape=jax.ShapeDtypeStruct((4, B_SZ, D), f32),
        compiler_params=pltpu.CompilerParams(
            vmem_limit_bytes=64 * 1024 * 1024),
    )(q, p, e, qv, params['type_embed'],
      wlr1h, wlr1l, a1h, a1l, b1m,
      wlr2h, wlr2l, w1h, w1l, w2h, w2l,
      v768, v3072)
    return out.transpose(1, 0, 2)


# speedup vs baseline: 1.1861x; 1.0371x over previous
import jax
import jax.numpy as jnp
import numpy as np
from jax.experimental import pallas as pl
from jax.experimental.pallas import tpu as pltpu

B_SZ = 4096
D = 768
H = 4
OC1 = 192
BT = 128
HB = BT // 2

_EDGES = [(0, 1, 0), (1, 0, 0), (0, 2, 0), (2, 0, 0), (0, 3, 0), (3, 0, 0),
          (1, 2, 1), (2, 1, 1), (1, 3, 1), (3, 1, 1), (2, 3, 1), (3, 2, 1)]
_INC = {d: [(s, wk) for (s, dd, wk) in _EDGES if dd == d] for d in range(4)}


def _bdot(a, b):
    return jax.lax.dot_general(a, b, (((1,), (0,)), ((), ())),
                               preferred_element_type=jnp.float32)


def _split(a):
    hi = a.astype(jnp.bfloat16)
    lo = (a - hi.astype(jnp.float32)).astype(jnp.bfloat16)
    return hi, lo


def _dot3(a, b_hi, b_lo):
    a_hi, a_lo = _split(a)
    return (_bdot(a_hi, b_hi) + _bdot(a_lo, b_hi)) + _bdot(a_hi, b_lo)


def _softmax3(logits):
    amax = jnp.maximum(jnp.maximum(logits[0], logits[1]), logits[2])
    exps = [jnp.exp(l - amax) for l in logits]
    den = exps[0] + exps[1] + exps[2] + 1e-16
    return [ex / den for ex in exps]


class _P:
    pass


def _attn1(p, xs, xlr):
    xln = [xlr[n * HB:(n + 1) * HB, 0:D] + p.bl1 for n in range(4)]
    yrn = [[xlr[n * HB:(n + 1) * HB, D:2 * D] + p.br1 + p.ea1[wk]
            for wk in range(2)] for n in range(4)]
    x2n = []
    for dn in range(4):
        logits, srcs = [], []
        for (s, wk) in _INC[dn]:
            m = xln[s] + yrn[dn][wk]
            m = jnp.where(m > 0, m, 0.2 * m)
            logits.append(_dot3(m, p.a1h, p.a1l))
            srcs.append(s)
        alphas = _softmax3(logits)
        outd = None
        for al, s in zip(alphas, srcs):
            a_hi, a_lo = _split(al)
            af = _bdot(a_hi, p.b1m) + _bdot(a_lo, p.b1m)
            t = af * xln[s]
            outd = t if outd is None else outd + t
        r = xs[dn] + outd + p.bias1
        x2n.append(jnp.where(r > 0, r, jnp.exp(r) - 1.0))
    return x2n


def _attn2(p, x2n, xlr2):
    acc = [None] * 4
    for h in range(4):
        o = h * D
        xl2n = [xlr2[n * HB:(n + 1) * HB, o:o + D] + p.bl2[h] for n in range(4)]
        yr2n = [[xlr2[n * HB:(n + 1) * HB, 4 * D + o:4 * D + o + D]
                 + p.br2[h] + p.ea2[wk][h] for wk in range(2)]
                for n in range(4)]
        att2h = p.att2[h]
        for dn in range(4):
            logits, srcs = [], []
            for (s, wk) in _INC[dn]:
                m = xl2n[s] + yr2n[dn][wk]
                m = jnp.where(m > 0, m, 0.2 * m)
                logits.append(jnp.sum(m * att2h, axis=1, keepdims=True))
                srcs.append(s)
            alphas = _softmax3(logits)
            outd = None
            for al, s in zip(alphas, srcs):
                t = al * xl2n[s]
                outd = t if outd is None else outd + t
            acc[dn] = outd if acc[dn] is None else acc[dn] + outd
    return jnp.concatenate(
        [x2n[dn] + 0.25 * acc[dn] + p.bias2 for dn in range(4)], axis=0)


def _mlp(p, x3):
    hmid = _dot3(x3, p.w1h, p.w1l) + p.b1v
    mu = jnp.mean(hmid, axis=1, keepdims=True)
    var = jnp.mean((hmid - mu) ** 2, axis=1, keepdims=True)
    hmid = (hmid - mu) * jax.lax.rsqrt(var + 1e-5) * p.ln_g + p.ln_b
    hmid = jnp.maximum(hmid, 0.0)
    return _dot3(hmid, p.w2h, p.w2l) + p.b2v


def _body(q_ref, p_ref, e_ref, qv_ref, te_ref,
          wlr1h_ref, wlr1l_ref, a1h_ref, a1l_ref, b1m_ref,
          wlr2h_ref, wlr2l_ref, w1h_ref, w1l_ref, w2h_ref, w2l_ref,
          v768_ref, v3072_ref, out_ref):
    p = _P()
    p.bl1 = v768_ref[0:1, :]
    p.br1 = v768_ref[1:2, :]
    p.ea1 = [v768_ref[2:3, :], v768_ref[3:4, :]]
    p.bias1 = v768_ref[5:6, :]
    p.bias2 = v768_ref[6:7, :]
    p.b1v = v768_ref[7:8, :]
    p.ln_g = v768_ref[8:9, :]
    p.ln_b = v768_ref[9:10, :]
    p.b2v = v768_ref[10:11, :]
    p.bl2 = [v3072_ref[0:1, h * D:(h + 1) * D] for h in range(4)]
    p.br2 = [v3072_ref[1:2, h * D:(h + 1) * D] for h in range(4)]
    p.ea2 = [[v3072_ref[2 + wk:3 + wk, h * D:(h + 1) * D] for h in range(4)]
             for wk in range(2)]
    p.att2 = [v3072_ref[4:5, h * D:(h + 1) * D] for h in range(4)]
    p.a1h = a1h_ref[...]
    p.a1l = a1l_ref[...]
    p.b1m = b1m_ref[...]
    p.w1h = w1h_ref[...]
    p.w1l = w1l_ref[...]
    p.w2h = w2h_ref[...]
    p.w2l = w2l_ref[...]
    te = te_ref[...]

    def build(lo):
        xs = [q_ref[lo:lo + HB] + te[0:1],
              p_ref[lo:lo + HB] + te[1:2],
              e_ref[lo:lo + HB] + te[2:3],
              qv_ref[lo:lo + HB] + te[3:4]]
        return xs, jnp.concatenate(xs, axis=0)

    wlr1h, wlr1l = wlr1h_ref[...], wlr1l_ref[...]
    wlr2h, wlr2l = wlr2h_ref[...], wlr2l_ref[...]

    xsA, xA = build(0)
    xsB, xB = build(HB)
    xlrA = _dot3(xA, wlr1h, wlr1l)
    xlrB = _dot3(xB, wlr1h, wlr1l)
    x2nA = _attn1(p, xsA, xlrA)
    xlr2A = _dot3(jnp.concatenate(x2nA, axis=0), wlr2h, wlr2l)
    x2nB = _attn1(p, xsB, xlrB)
    xlr2B = _dot3(jnp.concatenate(x2nB, axis=0), wlr2h, wlr2l)
    x3A = _attn2(p, x2nA, xlr2A)
    outA = _mlp(p, x3A)
    x3B = _attn2(p, x2nB, xlr2B)
    outB = _mlp(p, x3B)
    for n in range(4):
        out_ref[0:HB, n, :] = outA[n * HB:(n + 1) * HB]
        out_ref[HB:BT, n, :] = outB[n * HB:(n + 1) * HB]


def kernel(q, p, e, qv, params):
    L = params['layers']
    w = jax.nn.sigmoid(params['weak_weight'])
    f32 = jnp.float32

    we1 = L[0]['We'].reshape(-1)
    att1 = L[0]['att'].reshape(-1)
    we2 = L[1]['We'].reshape(-1)
    att2 = L[1]['att'].reshape(-1)

    zeros = jnp.zeros((D,), f32)
    v768 = jnp.stack([
        L[0]['bl'], L[0]['br'], we1, w * we1, zeros,
        L[0]['bias'], L[1]['bias'], params['b1'], params['ln_g'],
        params['ln_b'], params['b2'], zeros, zeros, zeros, zeros, zeros,
    ]).astype(f32)
    z2 = jnp.zeros((4 * D,), f32)
    v3072 = jnp.stack([
        L[1]['bl'], L[1]['br'], we2, w * we2, att2, z2, z2, z2,
    ]).astype(f32)

    head_of_lane = jnp.arange(D) // OC1
    a1 = jnp.where(head_of_lane[:, None] == jnp.arange(128)[None, :],
                   att1[:, None], 0.0).astype(f32)
    b1m = (jnp.arange(128)[:, None] == head_of_lane[None, :]).astype(
        jnp.bfloat16)

    wlr1 = jnp.concatenate([L[0]['Wl'], L[0]['Wr']], axis=1)
    wlr2 = jnp.concatenate([L[1]['Wl'], L[1]['Wr']], axis=1)
    wlr1h, wlr1l = _split(wlr1)
    wlr2h, wlr2l = _split(wlr2)
    a1h, a1l = _split(a1)
    w1h, w1l = _split(params['W1'])
    w2h, w2l = _split(params['W2'])

    cspec = lambda shape: pl.BlockSpec(shape, lambda i: (0,) * len(shape))
    out = pl.pallas_call(
        _body,
        grid=(B_SZ // BT,),
        in_specs=[
            pl.BlockSpec((BT, D), lambda i: (i, 0)),
            pl.BlockSpec((BT, D), lambda i: (i, 0)),
            pl.BlockSpec((BT, D), lambda i: (i, 0)),
            pl.BlockSpec((BT, D), lambda i: (i, 0)),
            cspec((4, D)),
            cspec((D, 2 * D)),
            cspec((D, 2 * D)),
            cspec((D, 128)),
            cspec((D, 128)),
            cspec((128, D)),
            cspec((D, 8 * D)),
            cspec((D, 8 * D)),
            cspec((D, D)),
            cspec((D, D)),
            cspec((D, D)),
            cspec((D, D)),
            cspec((16, D)),
            cspec((8, 4 * D)),
        ],
        out_specs=pl.BlockSpec((BT, 4, D), lambda i: (i, 0, 0)),
        out_shape=jax.ShapeDtypeStruct((B_SZ, 4, D), f32),
        compiler_params=pltpu.CompilerParams(
            vmem_limit_bytes=64 * 1024 * 1024),
    )(q, p, e, qv, params['type_embed'],
      wlr1h, wlr1l, a1h, a1l, b1m,
      wlr2h, wlr2l, w1h, w1l, w2h, w2l,
      v768, v3072)
    return out
